# baseline (speedup 1.0000x reference)
import functools

import jax
import jax.numpy as jnp
from jax import lax
from jax.experimental import pallas as pl
from jax.experimental.pallas import tpu as pltpu
from jax.experimental.pallas import tpu_sc as plsc

_STAGES = 3
_N = 4096
_E = 131072
_TOT = _STAGES * _E
_PLANE = _N * _N

_NC, _NS = 2, 16
_NW = _NC * _NS
_EPW = _E // _NW
_CH = 128
_NCH = _EPW // _CH
_PR = _E // _CH


def _prep_body(alphas_ref, rows_ref, cols_ref, *out_refs):
    a = alphas_ref[...]
    m = jnp.max(a)
    e = jnp.exp(a - m)
    w = e * (1.0 / jnp.sum(e))
    r = rows_ref[...]
    c = cols_ref[...]
    idx = r * _N + c
    for s in range(_STAGES):
        out_refs[s][...] = w[s * _PR:(s + 1) * _PR]
        out_refs[_STAGES + s][...] = idx[s * _PR:(s + 1) * _PR]


_prep = pl.pallas_call(
    _prep_body,
    out_shape=(
        [jax.ShapeDtypeStruct((_PR, _CH), jnp.float32) for _ in range(_STAGES)]
        + [jax.ShapeDtypeStruct((_PR, _CH), jnp.int32) for _ in range(_STAGES)]
    ),
)


def _zero_body(o_ref):
    o_ref[...] = jnp.zeros_like(o_ref)


def _zero_dep_body(d_ref, o_ref):
    del d_ref
    o_ref[...] = jnp.zeros_like(o_ref)


_ZBLK = 2097152
_zeros = pl.pallas_call(
    _zero_dep_body,
    grid=(_PLANE // _ZBLK,),
    in_specs=[pl.BlockSpec(memory_space=pl.ANY)],
    out_specs=pl.BlockSpec((_ZBLK,), lambda i: (i,)),
    out_shape=jax.ShapeDtypeStruct((_PLANE,), jnp.float32),
)


@functools.partial(
    pl.kernel,
    mesh=plsc.VectorSubcoreMesh(core_axis_name="c", subcore_axis_name="s"),
    scratch_types=[
        pltpu.VMEM((_NCH, _CH), jnp.int32),
        pltpu.VMEM((_NCH, _CH), jnp.float32),
        pltpu.SemaphoreType.DMA,
    ],
    name="sc_scatter_overwrite",
)
def _sc_scatter(idx_hbm, w_hbm, out_ref, idx_v, w_v, sem):
    wid = lax.axis_index("s") * _NC + lax.axis_index("c")
    base = wid * _NCH
    pltpu.sync_copy(idx_hbm.at[pl.ds(base, _NCH)], idx_v)
    pltpu.sync_copy(w_hbm.at[pl.ds(base, _NCH)], w_v)

    def _fire(j, carry):
        pltpu.make_async_copy(w_v.at[j], out_ref.at[idx_v.at[j]], sem).start()
        return carry

    lax.fori_loop(0, _NCH, _fire, 0)

    def _drain(j, carry):
        pltpu.make_async_copy(w_v.at[0], out_ref.at[idx_v.at[0]], sem).wait()
        return carry

    lax.fori_loop(0, _NCH, _drain, 0)


_RBLK = 512
_OUT_SHAPE = jax.ShapeDtypeStruct((_STAGES, _N, _N), jnp.float32)


def _retile_first_body(i_ref, o_ref):
    o_ref[...] = i_ref[...].reshape(o_ref.shape)


def _retile_next_body(i_ref, big_ref, o_ref):
    del big_ref
    o_ref[...] = i_ref[...].reshape(o_ref.shape)


def _make_retile(s):
    in_spec = pl.BlockSpec((_RBLK * _N,), lambda j: (j,))
    out_spec = pl.BlockSpec((1, _RBLK, _N), lambda j: (s, j, 0))
    if s == 0:
        return pl.pallas_call(
            _retile_first_body,
            grid=(_N // _RBLK,),
            in_specs=[in_spec],
            out_specs=out_spec,
            out_shape=_OUT_SHAPE,
        )
    return pl.pallas_call(
        _retile_next_body,
        grid=(_N // _RBLK,),
        in_specs=[in_spec, pl.BlockSpec(memory_space=pl.ANY)],
        out_specs=out_spec,
        out_shape=_OUT_SHAPE,
        input_output_aliases={1: 0},
    )


_retiles = [_make_retile(s) for s in range(_STAGES)]


def kernel(alphas, idx_rows, idx_cols):
    a2 = alphas.reshape(_STAGES * _PR, _CH)
    r2 = idx_rows.reshape(_STAGES * _PR, _CH)
    c2 = idx_cols.reshape(_STAGES * _PR, _CH)
    outs = _prep(a2, r2, c2)
    ws, idxs = outs[:_STAGES], outs[_STAGES:]
    planes = []
    for s in range(_STAGES):
        ref = jax.new_ref(_zeros(ws[s]))
        _sc_scatter(idxs[s], ws[s], ref)
        planes.append(ref[...])
    big = _retiles[0](planes[0])
    for s in range(1, _STAGES):
        big = _retiles[s](planes[s], big)
    return big


# speedup vs baseline: 1.0003x; 1.0003x over previous
import functools

import jax
import jax.numpy as jnp
from jax import lax
from jax.experimental import pallas as pl
from jax.experimental.pallas import tpu as pltpu
from jax.experimental.pallas import tpu_sc as plsc

_STAGES = 3
_N = 4096
_E = 131072
_TOT = _STAGES * _E
_PLANE = _N * _N

_NC, _NS = 2, 16
_NW = _NC * _NS
_EPW = _E // _NW
_CH = 128
_NCH = _EPW // _CH
_PR = _E // _CH


def _prep_body(alphas_ref, rows_ref, cols_ref, *out_refs):
    a = alphas_ref[...]
    m = jnp.max(a)
    e = jnp.exp(a - m)
    w = e * (1.0 / jnp.sum(e))
    r = rows_ref[...]
    c = cols_ref[...]
    idx = r * _N + c
    for s in range(_STAGES):
        out_refs[s][...] = w[s * _PR:(s + 1) * _PR]
        out_refs[_STAGES + s][...] = idx[s * _PR:(s + 1) * _PR]


_prep = pl.pallas_call(
    _prep_body,
    out_shape=(
        [jax.ShapeDtypeStruct((_PR, _CH), jnp.float32) for _ in range(_STAGES)]
        + [jax.ShapeDtypeStruct((_PR, _CH), jnp.int32) for _ in range(_STAGES)]
    ),
)


def _zero_body(o_ref):
    o_ref[...] = jnp.zeros_like(o_ref)


_ZBLK = 2097152
_zeros = pl.pallas_call(
    _zero_body,
    grid=(_PLANE // _ZBLK,),
    out_specs=pl.BlockSpec((_ZBLK,), lambda i: (i,)),
    out_shape=jax.ShapeDtypeStruct((_PLANE,), jnp.float32),
)


def _make_sc_scatter(nplanes):
    @functools.partial(
        pl.kernel,
        mesh=plsc.VectorSubcoreMesh(core_axis_name="c", subcore_axis_name="s"),
        scratch_types=[
            pltpu.VMEM((nplanes * _NCH, _CH), jnp.int32),
            pltpu.VMEM((nplanes * _NCH, _CH), jnp.float32),
            pltpu.SemaphoreType.DMA,
        ],
        name=f"sc_scatter_overwrite_{nplanes}",
    )
    def _sc_scatter(*args):
        refs, (idx_v, w_v, sem) = args[:-3], args[-3:]
        idx_hbms = refs[0:nplanes]
        w_hbms = refs[nplanes:2 * nplanes]
        out_refs = refs[2 * nplanes:3 * nplanes]
        wid = lax.axis_index("s") * _NC + lax.axis_index("c")
        base = wid * _NCH
        for p in range(nplanes):
            pltpu.sync_copy(
                idx_hbms[p].at[pl.ds(base, _NCH)],
                idx_v.at[pl.ds(p * _NCH, _NCH)],
            )
            pltpu.sync_copy(
                w_hbms[p].at[pl.ds(base, _NCH)],
                w_v.at[pl.ds(p * _NCH, _NCH)],
            )

        for p in range(nplanes):
            out_ref = out_refs[p]

            def _fire(j, carry, p=p, out_ref=out_ref):
                jj = p * _NCH + j
                pltpu.make_async_copy(
                    w_v.at[jj], out_ref.at[idx_v.at[jj]], sem
                ).start()
                return carry

            lax.fori_loop(0, _NCH, _fire, 0)

        def _drain(j, carry):
            pltpu.make_async_copy(
                w_v.at[0], out_refs[0].at[idx_v.at[0]], sem
            ).wait()
            return carry

        lax.fori_loop(0, nplanes * _NCH, _drain, 0)

    return _sc_scatter


_sc_scatter2 = _make_sc_scatter(2)
_sc_scatter1 = _make_sc_scatter(1)


_RBLK = 512
_OUT_SHAPE = jax.ShapeDtypeStruct((_STAGES, _N, _N), jnp.float32)


def _retile_first_body(i_ref, o_ref):
    o_ref[...] = i_ref[...].reshape(o_ref.shape)


def _retile_next_body(i_ref, big_ref, o_ref):
    del big_ref
    o_ref[...] = i_ref[...].reshape(o_ref.shape)


def _make_retile(s):
    in_spec = pl.BlockSpec((_RBLK * _N,), lambda j: (j,))
    out_spec = pl.BlockSpec((1, _RBLK, _N), lambda j: (s, j, 0))
    if s == 0:
        return pl.pallas_call(
            _retile_first_body,
            grid=(_N // _RBLK,),
            in_specs=[in_spec],
            out_specs=out_spec,
            out_shape=_OUT_SHAPE,
        )
    return pl.pallas_call(
        _retile_next_body,
        grid=(_N // _RBLK,),
        in_specs=[in_spec, pl.BlockSpec(memory_space=pl.ANY)],
        out_specs=out_spec,
        out_shape=_OUT_SHAPE,
        input_output_aliases={1: 0},
    )


_retiles = [_make_retile(s) for s in range(_STAGES)]


def kernel(alphas, idx_rows, idx_cols):
    a2 = alphas.reshape(_STAGES * _PR, _CH)
    r2 = idx_rows.reshape(_STAGES * _PR, _CH)
    c2 = idx_cols.reshape(_STAGES * _PR, _CH)
    outs = _prep(a2, r2, c2)
    ws, idxs = outs[:_STAGES], outs[_STAGES:]
    refs = [jax.new_ref(_zeros()) for _ in range(_STAGES)]
    _sc_scatter2(idxs[0], idxs[1], ws[0], ws[1], refs[0], refs[1])
    _sc_scatter1(idxs[2], ws[2], refs[2])
    planes = [r[...] for r in refs]
    big = _retiles[0](planes[0])
    for s in range(1, _STAGES):
        big = _retiles[s](planes[s], big)
    return big


# speedup vs baseline: 1.0061x; 1.0058x over previous
import functools

import jax
import jax.numpy as jnp
from jax import lax
from jax.experimental import pallas as pl
from jax.experimental.pallas import tpu as pltpu
from jax.experimental.pallas import tpu_sc as plsc

_STAGES = 3
_N = 4096
_E = 131072
_TOT = _STAGES * _E
_PLANE = _N * _N

_NC, _NS = 2, 16
_NW = _NC * _NS
_EPW = _E // _NW
_CH = 128
_NCH = _EPW // _CH
_PR = _E // _CH


def _prep_body(alphas_ref, rows_ref, cols_ref, *out_refs):
    a = alphas_ref[...]
    m = jnp.max(a)
    e = jnp.exp(a - m)
    w = e * (1.0 / jnp.sum(e))
    r = rows_ref[...]
    c = cols_ref[...]
    idx = r * _N + c
    for s in range(_STAGES):
        out_refs[s][...] = w[s * _PR:(s + 1) * _PR]
        out_refs[_STAGES + s][...] = idx[s * _PR:(s + 1) * _PR]


_prep = pl.pallas_call(
    _prep_body,
    out_shape=(
        [jax.ShapeDtypeStruct((_PR, _CH), jnp.float32) for _ in range(_STAGES)]
        + [jax.ShapeDtypeStruct((_PR, _CH), jnp.int32) for _ in range(_STAGES)]
    ),
)


def _zero_body(o_ref):
    o_ref[...] = jnp.zeros_like(o_ref)


_ZBLK = 2097152
_zeros = pl.pallas_call(
    _zero_body,
    grid=(_PLANE // _ZBLK,),
    out_specs=pl.BlockSpec((_ZBLK,), lambda i: (i,)),
    out_shape=jax.ShapeDtypeStruct((_PLANE,), jnp.float32),
)


def _make_sc_scatter(nplanes):
    @functools.partial(
        pl.kernel,
        mesh=plsc.VectorSubcoreMesh(core_axis_name="c", subcore_axis_name="s"),
        scratch_types=[
            pltpu.VMEM((nplanes * _NCH, _CH), jnp.int32),
            pltpu.VMEM((nplanes * _NCH, _CH), jnp.float32),
            pltpu.SemaphoreType.DMA,
        ],
        name=f"sc_scatter_overwrite_{nplanes}",
    )
    def _sc_scatter(*args):
        refs, (idx_v, w_v, sem) = args[:-3], args[-3:]
        idx_hbms = refs[0:nplanes]
        w_hbms = refs[nplanes:2 * nplanes]
        out_refs = refs[2 * nplanes:3 * nplanes]
        wid = lax.axis_index("s") * _NC + lax.axis_index("c")
        base = wid * _NCH
        for p in range(nplanes):
            pltpu.sync_copy(
                idx_hbms[p].at[pl.ds(base, _NCH)],
                idx_v.at[pl.ds(p * _NCH, _NCH)],
            )
            pltpu.sync_copy(
                w_hbms[p].at[pl.ds(base, _NCH)],
                w_v.at[pl.ds(p * _NCH, _NCH)],
            )

        for p in range(nplanes):
            out_ref = out_refs[p]

            def _fire(j, carry, p=p, out_ref=out_ref):
                jj = p * _NCH + j
                pltpu.make_async_copy(
                    w_v.at[jj], out_ref.at[idx_v.at[jj]], sem
                ).start()
                return carry

            lax.fori_loop(0, _NCH, _fire, 0)

        def _drain(j, carry):
            pltpu.make_async_copy(
                w_v.at[0], out_refs[0].at[idx_v.at[0]], sem
            ).wait()
            return carry

        lax.fori_loop(0, nplanes * _NCH, _drain, 0)

    return _sc_scatter


_sc_scatter1 = _make_sc_scatter(1)


_RBLK = 512
_OUT_SHAPE = jax.ShapeDtypeStruct((_STAGES, _N, _N), jnp.float32)


def _retile_first_body(i_ref, o_ref):
    o_ref[...] = i_ref[...].reshape(o_ref.shape)


def _retile_next_body(i_ref, big_ref, o_ref):
    del big_ref
    o_ref[...] = i_ref[...].reshape(o_ref.shape)


def _make_retile(s):
    in_spec = pl.BlockSpec((_RBLK * _N,), lambda j: (j,))
    out_spec = pl.BlockSpec((1, _RBLK, _N), lambda j: (s, j, 0))
    if s == 0:
        return pl.pallas_call(
            _retile_first_body,
            grid=(_N // _RBLK,),
            in_specs=[in_spec],
            out_specs=out_spec,
            out_shape=_OUT_SHAPE,
        )
    return pl.pallas_call(
        _retile_next_body,
        grid=(_N // _RBLK,),
        in_specs=[in_spec, pl.BlockSpec(memory_space=pl.ANY)],
        out_specs=out_spec,
        out_shape=_OUT_SHAPE,
        input_output_aliases={1: 0},
    )


_retiles = [_make_retile(s) for s in range(_STAGES)]


def kernel(alphas, idx_rows, idx_cols):
    a2 = alphas.reshape(_STAGES * _PR, _CH)
    r2 = idx_rows.reshape(_STAGES * _PR, _CH)
    c2 = idx_cols.reshape(_STAGES * _PR, _CH)
    outs = _prep(a2, r2, c2)
    ws, idxs = outs[:_STAGES], outs[_STAGES:]
    planes = []
    for s in range(_STAGES):
        ref = jax.new_ref(_zeros())
        _sc_scatter1(idxs[s], ws[s], ref)
        planes.append(ref[...])
    big = _retiles[0](planes[0])
    for s in range(1, _STAGES):
        big = _retiles[s](planes[s], big)
    return big
